# baseline (speedup 1.0000x reference)
import functools

import jax
import jax.numpy as jnp
from jax import lax
from jax.experimental import pallas as pl
from jax.experimental.pallas import tpu as pltpu
from jax.experimental.pallas import tpu_sc as plsc

N = 10000
E = 320000
D_IN = 128
H = 128
OUT = 64
FUS = 128

NC = 2
NS = 16
CHUNK = 80
EP = 327680
EPW = EP // (NC * NS)
NCHUNK = EPW // CHUNK
NBLK = 4
BCH = NCHUNK // NBLK
NPAD = 10240
RPT = NPAD // NS

BLK = 1000
GRID = N // BLK


@functools.cache
def _deg_kernel_build():
    mesh = plsc.VectorSubcoreMesh(core_axis_name="c", subcore_axis_name="s")
    return functools.partial(
        pl.kernel,
        out_type=jax.ShapeDtypeStruct((NC * NPAD,), jnp.float32),
        mesh=mesh,
        scratch_types=[
            pltpu.VMEM((BCH, CHUNK), jnp.int32),
            pltpu.VMEM((CHUNK,), jnp.float32),
            pltpu.VMEM_SHARED((NPAD,), jnp.float32),
            pltpu.SemaphoreType.DMA,
        ],
    )(_deg_body)


def _deg_body(dst4_hbm, ones_hbm, zeros_hbm, out_hbm, dst_v, ones_v, acc_sh,
              ssem):
    c = lax.axis_index("c")
    s = lax.axis_index("s")
    row0 = s * RPT
    pltpu.sync_copy(zeros_hbm, acc_sh.at[pl.ds(row0, RPT)])
    pltpu.sync_copy(ones_hbm, ones_v)
    wid = c * NS + s
    plsc.subcore_barrier()

    def blk(b, carry):
        pltpu.sync_copy(dst4_hbm.at[wid, b], dst_v)

        def step(j, c2):
            k0 = j * 4
            descs = [
                pltpu.async_copy(ones_v, acc_sh.at[dst_v.at[k0 + i]], ssem,
                                 add=True)
                for i in range(4)
            ]
            for d in descs:
                d.wait()
            return c2

        lax.fori_loop(0, BCH // 4, step, 0)
        return carry

    lax.fori_loop(0, NBLK, blk, 0)
    plsc.subcore_barrier()
    pltpu.sync_copy(acc_sh.at[pl.ds(row0, RPT)],
                    out_hbm.at[pl.ds(c * NPAD + row0, RPT)])


@functools.cache
def _agg_kernel_build():
    mesh = plsc.VectorSubcoreMesh(core_axis_name="c", subcore_axis_name="s")
    return functools.partial(
        pl.kernel,
        out_type=jax.ShapeDtypeStruct((NC * NPAD, H), jnp.float32),
        mesh=mesh,
        scratch_types=[
            pltpu.VMEM((BCH * CHUNK,), jnp.int32),
            pltpu.VMEM((BCH, CHUNK), jnp.int32),
            pltpu.VMEM((CHUNK, H), jnp.float32),
            pltpu.VMEM((CHUNK, H), jnp.float32),
            pltpu.VMEM((CHUNK, H), jnp.float32),
            pltpu.VMEM((CHUNK, H), jnp.float32),
            pltpu.VMEM_SHARED((NPAD, H), jnp.float32),
            pltpu.SemaphoreType.DMA,
            pltpu.SemaphoreType.DMA,
            pltpu.SemaphoreType.DMA,
            pltpu.SemaphoreType.DMA,
            pltpu.SemaphoreType.DMA,
            pltpu.SemaphoreType.DMA,
            pltpu.SemaphoreType.DMA,
            pltpu.SemaphoreType.DMA,
        ],
    )(_agg_body)


def _agg_body(src4_hbm, dst4_hbm, hs_hbm, zeros_hbm, out_hbm,
              src_v, dst_v, r0, r1, r2, r3, acc_sh,
              ga0, ga1, ga2, ga3, sa0, sa1, sa2, sa3):
    rows = (r0, r1, r2, r3)
    gsem = (ga0, ga1, ga2, ga3)
    ssem = (sa0, sa1, sa2, sa3)
    c = lax.axis_index("c")
    s = lax.axis_index("s")
    row0 = s * RPT
    pltpu.sync_copy(zeros_hbm, acc_sh.at[pl.ds(row0, RPT)])
    wid = c * NS + s
    plsc.subcore_barrier()

    def gath(k, i):
        return pltpu.async_copy(
            hs_hbm.at[src_v.at[pl.ds(k * CHUNK, CHUNK)]], rows[i], gsem[i])

    def wait_gath(i):
        pltpu.make_async_copy(hs_hbm.at[src_v.at[pl.ds(0, CHUNK)]], rows[i],
                              gsem[i]).wait()

    def scat(k, i):
        return pltpu.async_copy(rows[i], acc_sh.at[dst_v.at[k]], ssem[i],
                                add=True)

    def wait_scat(i):
        pltpu.make_async_copy(rows[i], acc_sh.at[dst_v.at[0]], ssem[i]).wait()

    def blk(b, carry):
        pltpu.sync_copy(src4_hbm.at[wid, b], src_v)
        pltpu.sync_copy(dst4_hbm.at[wid, b], dst_v)
        gath(0, 0)
        gath(1, 1)
        wait_gath(0)
        scat(0, 0)
        gath(2, 2)
        wait_gath(1)
        scat(1, 1)
        gath(3, 3)
        wait_gath(2)
        scat(2, 2)
        wait_scat(0)
        gath(4, 0)
        wait_gath(3)
        scat(3, 3)
        wait_scat(1)
        gath(5, 1)

        def body(j, c2):
            k0 = 4 * j
            for i in range(4):
                k = k0 + i
                wait_gath(i)
                scat(k, i)
                wait_scat((i + 2) % 4)
                gath(k + 2, (i + 2) % 4)
            return c2

        lax.fori_loop(1, BCH // 4 - 1, body, 0)
        wait_gath(0)
        scat(BCH - 4, 0)
        wait_scat(2)
        gath(BCH - 2, 2)
        wait_gath(1)
        scat(BCH - 3, 1)
        wait_scat(3)
        gath(BCH - 1, 3)
        wait_gath(2)
        scat(BCH - 2, 2)
        wait_scat(0)
        wait_gath(3)
        scat(BCH - 1, 3)
        wait_scat(1)
        wait_scat(2)
        wait_scat(3)
        return carry

    lax.fori_loop(0, NBLK, blk, 0)
    plsc.subcore_barrier()
    pltpu.sync_copy(acc_sh.at[pl.ds(row0, RPT)],
                    out_hbm.at[pl.ds(c * NPAD + row0, RPT)])


def _dinv(dega, degb):
    return lax.rsqrt(dega[:, :1] + degb[:, :1] + 1.0)


def _tc1_body(x_ref, dega_ref, degb_ref, w_ref, out_ref):
    dinv = _dinv(dega_ref[...], degb_ref[...])
    out_ref[...] = jnp.dot(x_ref[...], w_ref[...],
                           preferred_element_type=jnp.float32) * dinv


def _tc2_body(a1a_ref, a1b_ref, hs1_ref, dega_ref, degb_ref, gb1_ref, gw2_ref,
              out_ref):
    dinv = _dinv(dega_ref[...], degb_ref[...])
    h1 = jnp.maximum((a1a_ref[...] + a1b_ref[...] + hs1_ref[...]) * dinv
                     + gb1_ref[...], 0.0)
    out_ref[...] = jnp.dot(h1, gw2_ref[...],
                           preferred_element_type=jnp.float32) * dinv


def _tc3_body(a2a_ref, a2b_ref, hs2_ref, dega_ref, degb_ref, fem_ref, topo_ref,
              gb2_ref, fw0a_ref, fw0b_ref, fb0_ref, fw1_ref, fb1_ref,
              fwo_ref, fbo_ref, tw0a_ref, tw0b_ref, tb0_ref, tw1_ref, tb1_ref,
              two_ref, tbo_ref, uwh_ref, uwf_ref, uwt_ref, ub_ref,
              rw1_ref, rb1_ref, rw2_ref, rb2_ref, aw1_ref, ab1_ref,
              aw2_ref, ab2_ref, add_ref, rem_ref):
    mm = functools.partial(jnp.dot, preferred_element_type=jnp.float32)
    dinv = _dinv(dega_ref[...], degb_ref[...])
    h = jnp.maximum((a2a_ref[...] + a2b_ref[...] + hs2_ref[...]) * dinv
                    + gb2_ref[...], 0.0)
    f = jnp.maximum(mm(h, fw0a_ref[...]) + fem_ref[...] * fw0b_ref[...]
                    + fb0_ref[...], 0.0)
    f = jnp.maximum(mm(f, fw1_ref[...]) + fb1_ref[...], 0.0)
    f = mm(f, fwo_ref[...]) + fbo_ref[...]
    t = jnp.maximum(mm(h, tw0a_ref[...]) + topo_ref[...] * tw0b_ref[...]
                    + tb0_ref[...], 0.0)
    t = jnp.maximum(mm(t, tw1_ref[...]) + tb1_ref[...], 0.0)
    t = mm(t, two_ref[...]) + tbo_ref[...]
    fus = jnp.maximum(mm(h, uwh_ref[...]) + mm(f, uwf_ref[...])
                      + mm(t, uwt_ref[...]) + ub_ref[...], 0.0)
    r = jnp.maximum(mm(fus, rw1_ref[...]) + rb1_ref[...], 0.0)
    rem_ref[...] = jnp.tanh(mm(r, rw2_ref[...]) + rb2_ref[...])
    a = jnp.maximum(mm(fus, aw1_ref[...]) + ab1_ref[...], 0.0)
    add_ref[...] = jnp.tanh(mm(a, aw2_ref[...]) + ab2_ref[...])


def _row_spec(width):
    return pl.BlockSpec((BLK, width), lambda i: (i, 0))


def _full_spec(shape):
    return pl.BlockSpec(shape, lambda i: (0,) * len(shape))


def _tc_call(body, ins, row_widths, full_shapes, out_widths):
    in_specs = ([_row_spec(w) for w in row_widths]
                + [_full_spec(s) for s in full_shapes])
    out_specs = [_row_spec(w) for w in out_widths]
    out_shape = [jax.ShapeDtypeStruct((N, w), jnp.float32) for w in out_widths]
    outs = pl.pallas_call(
        body,
        grid=(GRID,),
        in_specs=in_specs,
        out_specs=out_specs if len(out_specs) > 1 else out_specs[0],
        out_shape=out_shape if len(out_shape) > 1 else out_shape[0],
    )(*ins)
    return outs


def kernel(x, edge_index, fem_reward, topology_reward, gW1, gb1, gW2, gb2,
           fW0, fb0, fW1, fb1, fWo, fbo, tW0, tb0, tW1, tb1, tWo, tbo,
           uW, ub, rW1, rb1, rW2, rb2, aW1, ab1, aW2, ab2):
    srcp = jnp.concatenate(
        [edge_index[0], jnp.zeros((EP - E,), jnp.int32)])
    dstp = jnp.concatenate(
        [edge_index[1],
         N + (jnp.arange(EP - E, dtype=jnp.int32) % (NPAD - N))])
    src3 = srcp.reshape(NC * NS, NBLK, BCH * CHUNK)
    dst4 = dstp.reshape(NC * NS, NBLK, BCH, CHUNK)
    zeros_h = jnp.zeros((RPT, H), jnp.float32)
    zeros_d = jnp.zeros((RPT,), jnp.float32)
    ones_d = jnp.ones((CHUNK,), jnp.float32)

    deg_parts = _deg_kernel_build()(dst4, ones_d, zeros_d)
    dega = deg_parts[:N].reshape(N, 1)
    degb = deg_parts[NPAD:NPAD + N].reshape(N, 1)

    h1s = _tc_call(_tc1_body, (x, dega, degb, gW1), (D_IN, 1, 1),
                   ((D_IN, H),), (H,))
    agg1 = _agg_kernel_build()(src3, dst4, h1s, zeros_h)
    a1a = agg1[:N]
    a1b = agg1[NPAD:NPAD + N]

    h2s = _tc_call(_tc2_body,
                   (a1a, a1b, h1s, dega, degb, gb1.reshape(1, H), gW2),
                   (H, H, H, 1, 1), ((1, H), (H, H)), (H,))
    agg2 = _agg_kernel_build()(src3, dst4, h2s, zeros_h)
    a2a = agg2[:N]
    a2b = agg2[NPAD:NPAD + N]

    fW0a, fW0b = fW0[:H], fW0[H:H + 1]
    tW0a, tW0b = tW0[:H], tW0[H:H + 1]
    uWh, uWf, uWt = uW[:H], uW[H:H + OUT], uW[H + OUT:]
    rW2p = jnp.pad(rW2, ((0, 0), (0, 7)))
    rb2p = jnp.pad(rb2, (0, 7)).reshape(1, 8)
    aW2p = jnp.pad(aW2, ((0, 0), (0, 5)))
    ab2p = jnp.pad(ab2, (0, 5)).reshape(1, 8)

    add_p, rem_p = _tc_call(
        _tc3_body,
        (a2a, a2b, h2s, dega, degb, fem_reward, topology_reward,
         gb2.reshape(1, H),
         fW0a, fW0b, fb0.reshape(1, H), fW1, fb1.reshape(1, H),
         fWo, fbo.reshape(1, OUT),
         tW0a, tW0b, tb0.reshape(1, H), tW1, tb1.reshape(1, H),
         tWo, tbo.reshape(1, OUT),
         uWh, uWf, uWt, ub.reshape(1, FUS),
         rW1, rb1.reshape(1, FUS), rW2p, rb2p,
         aW1, ab1.reshape(1, FUS), aW2p, ab2p),
        (H, H, H, 1, 1, 1, 1),
        ((1, H), (H, H), (1, H), (1, H), (H, H), (1, H), (H, OUT), (1, OUT),
         (H, H), (1, H), (1, H), (H, H), (1, H), (H, OUT), (1, OUT),
         (H, FUS), (OUT, FUS), (OUT, FUS), (1, FUS),
         (FUS, FUS), (1, FUS), (FUS, 8), (1, 8),
         (FUS, FUS), (1, FUS), (FUS, 8), (1, 8)),
        (8, 8))
    return (add_p[:, :3], rem_p[:, :1])


# speedup vs baseline: 2.8099x; 2.8099x over previous
import functools

import jax
import jax.numpy as jnp
from jax import lax
from jax.experimental import pallas as pl
from jax.experimental.pallas import tpu as pltpu
from jax.experimental.pallas import tpu_sc as plsc

N = 10000
E = 320000
D_IN = 128
H = 128
OUT = 64
FUS = 128

NC = 2
NS = 16
CHUNK = 80
EPW = E // (NC * NS)
NCHUNK = EPW // CHUNK
NBLK = 5
BCH = NCHUNK // NBLK
NPAD = 10240
RPT = NPAD // NS

BLK = 1000
GRID = N // BLK


@functools.cache
def _deg_kernel_build():
    mesh = plsc.VectorSubcoreMesh(core_axis_name="c", subcore_axis_name="s")
    return functools.partial(
        pl.kernel,
        out_type=jax.ShapeDtypeStruct((NC * NPAD,), jnp.float32),
        mesh=mesh,
        scratch_types=[
            pltpu.VMEM((BCH, CHUNK), jnp.int32),
            pltpu.VMEM((CHUNK,), jnp.float32),
            pltpu.VMEM_SHARED((NPAD,), jnp.float32),
            pltpu.SemaphoreType.DMA,
        ],
    )(_deg_body)


def _deg_body(dst4_hbm, ones_hbm, zeros_hbm, out_hbm, dst_v, ones_v, acc_sh,
              ssem):
    c = lax.axis_index("c")
    s = lax.axis_index("s")
    row0 = s * RPT
    pltpu.sync_copy(zeros_hbm, acc_sh.at[pl.ds(row0, RPT)])
    pltpu.sync_copy(ones_hbm, ones_v)
    wid = c * NS + s
    plsc.subcore_barrier()

    def blk(b, carry):
        pltpu.sync_copy(dst4_hbm.at[wid, b], dst_v)

        def step(j, c2):
            k0 = j * 5
            descs = [
                pltpu.async_copy(ones_v, acc_sh.at[dst_v.at[k0 + i]], ssem,
                                 add=True)
                for i in range(5)
            ]
            for d in descs:
                d.wait()
            return c2

        lax.fori_loop(0, BCH // 5, step, 0)
        return carry

    lax.fori_loop(0, NBLK, blk, 0)
    plsc.subcore_barrier()
    pltpu.sync_copy(acc_sh.at[pl.ds(row0, RPT)],
                    out_hbm.at[pl.ds(c * NPAD + row0, RPT)])


@functools.cache
def _agg_kernel_build():
    mesh = plsc.VectorSubcoreMesh(core_axis_name="c", subcore_axis_name="s")
    return functools.partial(
        pl.kernel,
        out_type=jax.ShapeDtypeStruct((NC * NPAD, H), jnp.float32),
        mesh=mesh,
        scratch_types=[
            pltpu.VMEM((BCH * CHUNK,), jnp.int32),
            pltpu.VMEM((BCH, CHUNK), jnp.int32),
            pltpu.VMEM((CHUNK, H), jnp.float32),
            pltpu.VMEM((CHUNK, H), jnp.float32),
            pltpu.VMEM((CHUNK, H), jnp.float32),
            pltpu.VMEM((CHUNK, H), jnp.float32),
            pltpu.VMEM_SHARED((NPAD, H), jnp.float32),
            pltpu.SemaphoreType.DMA,
            pltpu.SemaphoreType.DMA,
            pltpu.SemaphoreType.DMA,
            pltpu.SemaphoreType.DMA,
            pltpu.SemaphoreType.DMA,
            pltpu.SemaphoreType.DMA,
            pltpu.SemaphoreType.DMA,
            pltpu.SemaphoreType.DMA,
        ],
    )(_agg_body)


def _agg_body(src4_hbm, dst4_hbm, hs_hbm, zeros_hbm, out_hbm,
              src_v, dst_v, r0, r1, r2, r3, acc_sh,
              ga0, ga1, ga2, ga3, sa0, sa1, sa2, sa3):
    rows = (r0, r1, r2, r3)
    gsem = (ga0, ga1, ga2, ga3)
    ssem = (sa0, sa1, sa2, sa3)
    c = lax.axis_index("c")
    s = lax.axis_index("s")
    row0 = s * RPT
    pltpu.sync_copy(zeros_hbm, acc_sh.at[pl.ds(row0, RPT)])
    wid = c * NS + s
    plsc.subcore_barrier()

    def gath(k, i):
        return pltpu.async_copy(
            hs_hbm.at[src_v.at[pl.ds(k * CHUNK, CHUNK)]], rows[i], gsem[i])

    def wait_gath(i):
        pltpu.make_async_copy(hs_hbm.at[src_v.at[pl.ds(0, CHUNK)]], rows[i],
                              gsem[i]).wait()

    def scat(k, i):
        return pltpu.async_copy(rows[i], acc_sh.at[dst_v.at[k]], ssem[i],
                                add=True)

    def wait_scat(i):
        pltpu.make_async_copy(rows[i], acc_sh.at[dst_v.at[0]], ssem[i]).wait()

    def blk(b, carry):
        pltpu.sync_copy(src4_hbm.at[wid, b], src_v)
        pltpu.sync_copy(dst4_hbm.at[wid, b], dst_v)
        gath(0, 0)
        gath(1, 1)
        wait_gath(0)
        scat(0, 0)
        gath(2, 2)
        wait_gath(1)
        scat(1, 1)
        gath(3, 3)
        wait_gath(2)
        scat(2, 2)
        wait_scat(0)
        gath(4, 0)
        wait_gath(3)
        scat(3, 3)
        wait_scat(1)
        gath(5, 1)

        def body(j, c2):
            k0 = 4 * j
            for i in range(4):
                k = k0 + i
                wait_gath(i)
                scat(k, i)
                wait_scat((i + 2) % 4)
                gath(k + 2, (i + 2) % 4)
            return c2

        lax.fori_loop(1, BCH // 4 - 1, body, 0)
        wait_gath(0)
        scat(20, 0)
        wait_scat(2)
        gath(22, 2)
        wait_gath(1)
        scat(21, 1)
        wait_scat(3)
        gath(23, 3)
        wait_gath(2)
        scat(22, 2)
        wait_scat(0)
        gath(24, 0)
        wait_gath(3)
        scat(23, 3)
        wait_scat(1)
        wait_gath(0)
        scat(24, 0)
        wait_scat(2)
        wait_scat(3)
        wait_scat(0)
        return carry

    lax.fori_loop(0, NBLK, blk, 0)
    plsc.subcore_barrier()
    pltpu.sync_copy(acc_sh.at[pl.ds(row0, RPT)],
                    out_hbm.at[pl.ds(c * NPAD + row0, RPT)])


def _dinv(dega, degb):
    return 1.0 / jnp.sqrt(dega[:, :1] + degb[:, :1] + 1.0)


def _tc1_body(x_ref, dega_ref, degb_ref, w_ref, out_ref):
    dinv = _dinv(dega_ref[...], degb_ref[...])
    out_ref[...] = jnp.dot(x_ref[...], w_ref[...],
                           preferred_element_type=jnp.float32) * dinv


def _tc2_body(a1a_ref, a1b_ref, hs1_ref, dega_ref, degb_ref, gb1_ref, gw2_ref,
              out_ref):
    dinv = _dinv(dega_ref[...], degb_ref[...])
    h1 = jnp.maximum((a1a_ref[...] + a1b_ref[...] + hs1_ref[...]) * dinv
                     + gb1_ref[...], 0.0)
    out_ref[...] = jnp.dot(h1, gw2_ref[...],
                           preferred_element_type=jnp.float32) * dinv


def _tc3_body(a2a_ref, a2b_ref, hs2_ref, dega_ref, degb_ref, fem_ref, topo_ref,
              gb2_ref, fw0a_ref, fw0b_ref, fb0_ref, fw1_ref, fb1_ref,
              fwo_ref, fbo_ref, tw0a_ref, tw0b_ref, tb0_ref, tw1_ref, tb1_ref,
              two_ref, tbo_ref, uwh_ref, uwf_ref, uwt_ref, ub_ref,
              rw1_ref, rb1_ref, rw2_ref, rb2_ref, aw1_ref, ab1_ref,
              aw2_ref, ab2_ref, add_ref, rem_ref):
    mm = functools.partial(jnp.dot, preferred_element_type=jnp.float32)
    dinv = _dinv(dega_ref[...], degb_ref[...])
    h = jnp.maximum((a2a_ref[...] + a2b_ref[...] + hs2_ref[...]) * dinv
                    + gb2_ref[...], 0.0)
    def _bf(v):
        return v.astype(jnp.bfloat16).astype(jnp.float32)

    f = jnp.maximum(mm(h, fw0a_ref[...]) + _bf(fem_ref[...]) * _bf(fw0b_ref[...])
                    + fb0_ref[...], 0.0)
    f = jnp.maximum(mm(f, fw1_ref[...]) + fb1_ref[...], 0.0)
    f = mm(f, fwo_ref[...]) + fbo_ref[...]
    t = jnp.maximum(mm(h, tw0a_ref[...]) + _bf(topo_ref[...]) * _bf(tw0b_ref[...])
                    + tb0_ref[...], 0.0)
    t = jnp.maximum(mm(t, tw1_ref[...]) + tb1_ref[...], 0.0)
    t = mm(t, two_ref[...]) + tbo_ref[...]
    fus = jnp.maximum(mm(h, uwh_ref[...]) + mm(f, uwf_ref[...])
                      + mm(t, uwt_ref[...]) + ub_ref[...], 0.0)
    r = jnp.maximum(mm(fus, rw1_ref[...]) + rb1_ref[...], 0.0)
    rem_ref[...] = jnp.tanh(mm(r, rw2_ref[...]) + rb2_ref[...])
    a = jnp.maximum(mm(fus, aw1_ref[...]) + ab1_ref[...], 0.0)
    add_ref[...] = jnp.tanh(mm(a, aw2_ref[...]) + ab2_ref[...])


def _row_spec(width):
    return pl.BlockSpec((BLK, width), lambda i: (i, 0))


def _full_spec(shape):
    return pl.BlockSpec(shape, lambda i: (0,) * len(shape))


def _tc_call(body, ins, row_widths, full_shapes, out_widths):
    in_specs = ([_row_spec(w) for w in row_widths]
                + [_full_spec(s) for s in full_shapes])
    out_specs = [_row_spec(w) for w in out_widths]
    out_shape = [jax.ShapeDtypeStruct((N, w), jnp.float32) for w in out_widths]
    outs = pl.pallas_call(
        body,
        grid=(GRID,),
        in_specs=in_specs,
        out_specs=out_specs if len(out_specs) > 1 else out_specs[0],
        out_shape=out_shape if len(out_shape) > 1 else out_shape[0],
    )(*ins)
    return outs


def kernel(x, edge_index, fem_reward, topology_reward, gW1, gb1, gW2, gb2,
           fW0, fb0, fW1, fb1, fWo, fbo, tW0, tb0, tW1, tb1, tWo, tbo,
           uW, ub, rW1, rb1, rW2, rb2, aW1, ab1, aW2, ab2):
    src3 = edge_index[0].reshape(NC * NS, NBLK, BCH * CHUNK)
    dst4 = edge_index[1].reshape(NC * NS, NBLK, BCH, CHUNK)
    zeros_h = jnp.zeros((RPT, H), jnp.float32)
    zeros_d = jnp.zeros((RPT,), jnp.float32)
    ones_d = jnp.ones((CHUNK,), jnp.float32)

    deg_parts = _deg_kernel_build()(dst4, ones_d, zeros_d)
    dega = deg_parts[:N].reshape(N, 1)
    degb = deg_parts[NPAD:NPAD + N].reshape(N, 1)

    h1s = _tc_call(_tc1_body, (x, dega, degb, gW1), (D_IN, 1, 1),
                   ((D_IN, H),), (H,))
    agg1 = _agg_kernel_build()(src3, dst4, h1s, zeros_h)
    a1a = agg1[:N]
    a1b = agg1[NPAD:NPAD + N]

    h2s = _tc_call(_tc2_body,
                   (a1a, a1b, h1s, dega, degb, gb1.reshape(1, H), gW2),
                   (H, H, H, 1, 1), ((1, H), (H, H)), (H,))
    agg2 = _agg_kernel_build()(src3, dst4, h2s, zeros_h)
    a2a = agg2[:N]
    a2b = agg2[NPAD:NPAD + N]

    fW0a, fW0b = fW0[:H], fW0[H:H + 1]
    tW0a, tW0b = tW0[:H], tW0[H:H + 1]
    uWh, uWf, uWt = uW[:H], uW[H:H + OUT], uW[H + OUT:]
    rW2p = jnp.pad(rW2, ((0, 0), (0, 7)))
    rb2p = jnp.pad(rb2, (0, 7)).reshape(1, 8)
    aW2p = jnp.pad(aW2, ((0, 0), (0, 5)))
    ab2p = jnp.pad(ab2, (0, 5)).reshape(1, 8)

    add_p, rem_p = _tc_call(
        _tc3_body,
        (a2a, a2b, h2s, dega, degb, fem_reward, topology_reward,
         gb2.reshape(1, H),
         fW0a, fW0b, fb0.reshape(1, H), fW1, fb1.reshape(1, H),
         fWo, fbo.reshape(1, OUT),
         tW0a, tW0b, tb0.reshape(1, H), tW1, tb1.reshape(1, H),
         tWo, tbo.reshape(1, OUT),
         uWh, uWf, uWt, ub.reshape(1, FUS),
         rW1, rb1.reshape(1, FUS), rW2p, rb2p,
         aW1, ab1.reshape(1, FUS), aW2p, ab2p),
        (H, H, H, 1, 1, 1, 1),
        ((1, H), (H, H), (1, H), (1, H), (H, H), (1, H), (H, OUT), (1, OUT),
         (H, H), (1, H), (1, H), (H, H), (1, H), (H, OUT), (1, OUT),
         (H, FUS), (OUT, FUS), (OUT, FUS), (1, FUS),
         (FUS, FUS), (1, FUS), (FUS, 8), (1, 8),
         (FUS, FUS), (1, FUS), (FUS, 8), (1, 8)),
        (8, 8))
    return (add_p[:, :3], rem_p[:, :1])


# speedup vs baseline: 2.8580x; 1.0171x over previous
import functools

import jax
import jax.numpy as jnp
from jax import lax
from jax.experimental import pallas as pl
from jax.experimental.pallas import tpu as pltpu
from jax.experimental.pallas import tpu_sc as plsc

N = 10000
E = 320000
D_IN = 128
H = 128
OUT = 64
FUS = 128

NC = 2
NS = 16
CHUNK = 80
EPW = E // (NC * NS)
NCHUNK = EPW // CHUNK
NBLK = 5
BCH = NCHUNK // NBLK
NPAD = 10240
RPT = NPAD // NS

BLK = 2000
GRID = N // BLK


@functools.cache
def _deg_kernel_build():
    mesh = plsc.VectorSubcoreMesh(core_axis_name="c", subcore_axis_name="s")
    return functools.partial(
        pl.kernel,
        out_type=jax.ShapeDtypeStruct((NC * NPAD,), jnp.float32),
        mesh=mesh,
        scratch_types=[
            pltpu.VMEM((BCH, CHUNK), jnp.int32),
            pltpu.VMEM((CHUNK,), jnp.float32),
            pltpu.VMEM_SHARED((NPAD,), jnp.float32),
            pltpu.SemaphoreType.DMA,
        ],
    )(_deg_body)


def _deg_body(dst4_hbm, ones_hbm, zeros_hbm, out_hbm, dst_v, ones_v, acc_sh,
              ssem):
    c = lax.axis_index("c")
    s = lax.axis_index("s")
    row0 = s * RPT
    pltpu.sync_copy(zeros_hbm, acc_sh.at[pl.ds(row0, RPT)])
    pltpu.sync_copy(ones_hbm, ones_v)
    wid = c * NS + s
    plsc.subcore_barrier()

    def blk(b, carry):
        pltpu.sync_copy(dst4_hbm.at[wid, b], dst_v)

        def step(j, c2):
            k0 = j * 5
            descs = [
                pltpu.async_copy(ones_v, acc_sh.at[dst_v.at[k0 + i]], ssem,
                                 add=True)
                for i in range(5)
            ]
            for d in descs:
                d.wait()
            return c2

        lax.fori_loop(0, BCH // 5, step, 0)
        return carry

    lax.fori_loop(0, NBLK, blk, 0)
    plsc.subcore_barrier()
    pltpu.sync_copy(acc_sh.at[pl.ds(row0, RPT)],
                    out_hbm.at[pl.ds(c * NPAD + row0, RPT)])


@functools.cache
def _agg_kernel_build():
    mesh = plsc.VectorSubcoreMesh(core_axis_name="c", subcore_axis_name="s")
    return functools.partial(
        pl.kernel,
        out_type=jax.ShapeDtypeStruct((NC * NPAD, H), jnp.float32),
        mesh=mesh,
        scratch_types=[
            pltpu.VMEM((BCH * CHUNK,), jnp.int32),
            pltpu.VMEM((BCH, CHUNK), jnp.int32),
            pltpu.VMEM((CHUNK, H), jnp.float32),
            pltpu.VMEM((CHUNK, H), jnp.float32),
            pltpu.VMEM((CHUNK, H), jnp.float32),
            pltpu.VMEM((CHUNK, H), jnp.float32),
            pltpu.VMEM_SHARED((NPAD, H), jnp.float32),
            pltpu.SemaphoreType.DMA,
            pltpu.SemaphoreType.DMA,
            pltpu.SemaphoreType.DMA,
            pltpu.SemaphoreType.DMA,
            pltpu.SemaphoreType.DMA,
            pltpu.SemaphoreType.DMA,
            pltpu.SemaphoreType.DMA,
            pltpu.SemaphoreType.DMA,
        ],
    )(_agg_body)


def _agg_body(src4_hbm, dst4_hbm, hs_hbm, zeros_hbm, out_hbm,
              src_v, dst_v, r0, r1, r2, r3, acc_sh,
              ga0, ga1, ga2, ga3, sa0, sa1, sa2, sa3):
    rows = (r0, r1, r2, r3)
    gsem = (ga0, ga1, ga2, ga3)
    ssem = (sa0, sa1, sa2, sa3)
    c = lax.axis_index("c")
    s = lax.axis_index("s")
    row0 = s * RPT
    pltpu.sync_copy(zeros_hbm, acc_sh.at[pl.ds(row0, RPT)])
    wid = c * NS + s
    plsc.subcore_barrier()

    def gath(k, i):
        return pltpu.async_copy(
            hs_hbm.at[src_v.at[pl.ds(k * CHUNK, CHUNK)]], rows[i], gsem[i])

    def wait_gath(i):
        pltpu.make_async_copy(hs_hbm.at[src_v.at[pl.ds(0, CHUNK)]], rows[i],
                              gsem[i]).wait()

    def scat(k, i):
        return pltpu.async_copy(rows[i], acc_sh.at[dst_v.at[k]], ssem[i],
                                add=True)

    def wait_scat(i):
        pltpu.make_async_copy(rows[i], acc_sh.at[dst_v.at[0]], ssem[i]).wait()

    def blk(b, carry):
        pltpu.sync_copy(src4_hbm.at[wid, b], src_v)
        pltpu.sync_copy(dst4_hbm.at[wid, b], dst_v)
        gath(0, 0)
        gath(1, 1)
        wait_gath(0)
        scat(0, 0)
        gath(2, 2)
        wait_gath(1)
        scat(1, 1)
        gath(3, 3)
        wait_gath(2)
        scat(2, 2)
        wait_scat(0)
        gath(4, 0)
        wait_gath(3)
        scat(3, 3)
        wait_scat(1)
        gath(5, 1)

        def body(j, c2):
            k0 = 4 * j
            for i in range(4):
                k = k0 + i
                wait_gath(i)
                scat(k, i)
                wait_scat((i + 2) % 4)
                gath(k + 2, (i + 2) % 4)
            return c2

        lax.fori_loop(1, BCH // 4 - 1, body, 0)
        wait_gath(0)
        scat(20, 0)
        wait_scat(2)
        gath(22, 2)
        wait_gath(1)
        scat(21, 1)
        wait_scat(3)
        gath(23, 3)
        wait_gath(2)
        scat(22, 2)
        wait_scat(0)
        gath(24, 0)
        wait_gath(3)
        scat(23, 3)
        wait_scat(1)
        wait_gath(0)
        scat(24, 0)
        wait_scat(2)
        wait_scat(3)
        wait_scat(0)
        return carry

    lax.fori_loop(0, NBLK, blk, 0)
    plsc.subcore_barrier()
    pltpu.sync_copy(acc_sh.at[pl.ds(row0, RPT)],
                    out_hbm.at[pl.ds(c * NPAD + row0, RPT)])


def _dinv(dega, degb):
    return 1.0 / jnp.sqrt(dega[:, :1] + degb[:, :1] + 1.0)


def _tc1_body(x_ref, dega_ref, degb_ref, w_ref, out_ref):
    dinv = _dinv(dega_ref[...], degb_ref[...])
    out_ref[...] = jnp.dot(x_ref[...], w_ref[...],
                           preferred_element_type=jnp.float32) * dinv


def _tc2_body(a1a_ref, a1b_ref, hs1_ref, dega_ref, degb_ref, gb1_ref, gw2_ref,
              out_ref):
    dinv = _dinv(dega_ref[...], degb_ref[...])
    h1 = jnp.maximum((a1a_ref[...] + a1b_ref[...] + hs1_ref[...]) * dinv
                     + gb1_ref[...], 0.0)
    out_ref[...] = jnp.dot(h1, gw2_ref[...],
                           preferred_element_type=jnp.float32) * dinv


def _tc3_body(a2a_ref, a2b_ref, hs2_ref, dega_ref, degb_ref, fem_ref, topo_ref,
              gb2_ref, fw0a_ref, fw0b_ref, fb0_ref, fw1_ref, fb1_ref,
              fwo_ref, fbo_ref, tw0a_ref, tw0b_ref, tb0_ref, tw1_ref, tb1_ref,
              two_ref, tbo_ref, uwh_ref, uwf_ref, uwt_ref, ub_ref,
              rw1_ref, rb1_ref, rw2_ref, rb2_ref, aw1_ref, ab1_ref,
              aw2_ref, ab2_ref, add_ref, rem_ref):
    mm = functools.partial(jnp.dot, preferred_element_type=jnp.float32)
    dinv = _dinv(dega_ref[...], degb_ref[...])
    h = jnp.maximum((a2a_ref[...] + a2b_ref[...] + hs2_ref[...]) * dinv
                    + gb2_ref[...], 0.0)
    def _bf(v):
        return v.astype(jnp.bfloat16).astype(jnp.float32)

    f = jnp.maximum(mm(h, fw0a_ref[...]) + _bf(fem_ref[...]) * _bf(fw0b_ref[...])
                    + fb0_ref[...], 0.0)
    f = jnp.maximum(mm(f, fw1_ref[...]) + fb1_ref[...], 0.0)
    f = mm(f, fwo_ref[...]) + fbo_ref[...]
    t = jnp.maximum(mm(h, tw0a_ref[...]) + _bf(topo_ref[...]) * _bf(tw0b_ref[...])
                    + tb0_ref[...], 0.0)
    t = jnp.maximum(mm(t, tw1_ref[...]) + tb1_ref[...], 0.0)
    t = mm(t, two_ref[...]) + tbo_ref[...]
    fus = jnp.maximum(mm(h, uwh_ref[...]) + mm(f, uwf_ref[...])
                      + mm(t, uwt_ref[...]) + ub_ref[...], 0.0)
    r = jnp.maximum(mm(fus, rw1_ref[...]) + rb1_ref[...], 0.0)
    rem_ref[...] = jnp.tanh(mm(r, rw2_ref[...]) + rb2_ref[...])
    a = jnp.maximum(mm(fus, aw1_ref[...]) + ab1_ref[...], 0.0)
    add_ref[...] = jnp.tanh(mm(a, aw2_ref[...]) + ab2_ref[...])


def _row_spec(width):
    return pl.BlockSpec((BLK, width), lambda i: (i, 0))


def _full_spec(shape):
    return pl.BlockSpec(shape, lambda i: (0,) * len(shape))


def _tc_call(body, ins, row_widths, full_shapes, out_widths):
    in_specs = ([_row_spec(w) for w in row_widths]
                + [_full_spec(s) for s in full_shapes])
    out_specs = [_row_spec(w) for w in out_widths]
    out_shape = [jax.ShapeDtypeStruct((N, w), jnp.float32) for w in out_widths]
    outs = pl.pallas_call(
        body,
        grid=(GRID,),
        in_specs=in_specs,
        out_specs=out_specs if len(out_specs) > 1 else out_specs[0],
        out_shape=out_shape if len(out_shape) > 1 else out_shape[0],
    )(*ins)
    return outs


def kernel(x, edge_index, fem_reward, topology_reward, gW1, gb1, gW2, gb2,
           fW0, fb0, fW1, fb1, fWo, fbo, tW0, tb0, tW1, tb1, tWo, tbo,
           uW, ub, rW1, rb1, rW2, rb2, aW1, ab1, aW2, ab2):
    src3 = edge_index[0].reshape(NC * NS, NBLK, BCH * CHUNK)
    dst4 = edge_index[1].reshape(NC * NS, NBLK, BCH, CHUNK)
    zeros_h = jnp.zeros((RPT, H), jnp.float32)
    zeros_d = jnp.zeros((RPT,), jnp.float32)
    ones_d = jnp.ones((CHUNK,), jnp.float32)

    deg_parts = _deg_kernel_build()(dst4, ones_d, zeros_d)
    dega = deg_parts[:N].reshape(N, 1)
    degb = deg_parts[NPAD:NPAD + N].reshape(N, 1)

    h1s = _tc_call(_tc1_body, (x, dega, degb, gW1), (D_IN, 1, 1),
                   ((D_IN, H),), (H,))
    agg1 = _agg_kernel_build()(src3, dst4, h1s, zeros_h)
    a1a = agg1[:N]
    a1b = agg1[NPAD:NPAD + N]

    h2s = _tc_call(_tc2_body,
                   (a1a, a1b, h1s, dega, degb, gb1.reshape(1, H), gW2),
                   (H, H, H, 1, 1), ((1, H), (H, H)), (H,))
    agg2 = _agg_kernel_build()(src3, dst4, h2s, zeros_h)
    a2a = agg2[:N]
    a2b = agg2[NPAD:NPAD + N]

    fW0a, fW0b = fW0[:H], fW0[H:H + 1]
    tW0a, tW0b = tW0[:H], tW0[H:H + 1]
    uWh, uWf, uWt = uW[:H], uW[H:H + OUT], uW[H + OUT:]
    rW2p = jnp.pad(rW2, ((0, 0), (0, 7)))
    rb2p = jnp.pad(rb2, (0, 7)).reshape(1, 8)
    aW2p = jnp.pad(aW2, ((0, 0), (0, 5)))
    ab2p = jnp.pad(ab2, (0, 5)).reshape(1, 8)

    add_p, rem_p = _tc_call(
        _tc3_body,
        (a2a, a2b, h2s, dega, degb, fem_reward, topology_reward,
         gb2.reshape(1, H),
         fW0a, fW0b, fb0.reshape(1, H), fW1, fb1.reshape(1, H),
         fWo, fbo.reshape(1, OUT),
         tW0a, tW0b, tb0.reshape(1, H), tW1, tb1.reshape(1, H),
         tWo, tbo.reshape(1, OUT),
         uWh, uWf, uWt, ub.reshape(1, FUS),
         rW1, rb1.reshape(1, FUS), rW2p, rb2p,
         aW1, ab1.reshape(1, FUS), aW2p, ab2p),
        (H, H, H, 1, 1, 1, 1),
        ((1, H), (H, H), (1, H), (1, H), (H, H), (1, H), (H, OUT), (1, OUT),
         (H, H), (1, H), (1, H), (H, H), (1, H), (H, OUT), (1, OUT),
         (H, FUS), (OUT, FUS), (OUT, FUS), (1, FUS),
         (FUS, FUS), (1, FUS), (FUS, 8), (1, 8),
         (FUS, FUS), (1, FUS), (FUS, 8), (1, 8)),
        (8, 8))
    return (add_p[:, :3], rem_p[:, :1])
